# baseline (speedup 1.0000x reference)
import jax
import jax.numpy as jnp
from jax import lax
from jax.experimental import pallas as pl
from jax.experimental.pallas import tpu as pltpu
from jax.experimental.pallas import tpu_sc as plsc

B = 16384
ED = 64
RD = 32
NW = 32
PW = B // NW
HP = PW // 2
CH = 16
NCH = HP // CH


def _sc_body(head_hbm, rel_hbm, tail_hbm, ent_hbm, relemb_hbm, projt_hbm,
             out_hbm, hbuf, tbuf, rebuf, obuf, pb0, pb1, hidx, tidx, ridx,
             sem_g, sem_p0, sem_p1):
  wid = lax.axis_index("s") * 2 + lax.axis_index("c")
  for half in range(2):
    base = wid * PW + half * HP
    r0 = wid * 4 + half * 2

    pltpu.sync_copy(head_hbm.at[pl.ds(r0, 2)], hidx)
    pltpu.sync_copy(tail_hbm.at[pl.ds(r0, 2)], tidx)
    pltpu.sync_copy(rel_hbm.at[pl.ds(r0, 2)], ridx)

    gathers = []
    for c in range(2):
      gathers.append(pltpu.make_async_copy(
          ent_hbm.at[hidx.at[c]], hbuf.at[pl.ds(c * 128, 128)], sem_g))
      gathers.append(pltpu.make_async_copy(
          ent_hbm.at[tidx.at[c]], tbuf.at[pl.ds(c * 128, 128)], sem_g))
      gathers.append(pltpu.make_async_copy(
          relemb_hbm.at[ridx.at[c]], rebuf.at[pl.ds(c * 128, 128)], sem_g))
    for g in gathers:
      g.start()

    def p_desc(c, buf, sem):
      row = c // 8
      col = (c % 8) * CH
      return pltpu.make_async_copy(
          projt_hbm.at[ridx.at[row, pl.ds(col, CH)]], buf, sem)

    p_desc(0, pb0, sem_p0).start()
    p_desc(1, pb1, sem_p1).start()

    for g in gathers:
      g.wait()

    def dsub(i, carry):
      b = i // 4
      k = (i % 4) * 16
      hbuf[b, pl.ds(k, 16)] = hbuf[b, pl.ds(k, 16)] - tbuf[b, pl.ds(k, 16)]
      return carry
    lax.fori_loop(0, HP * 4, dsub, 0)

    def ring(it, carry):
      for bb, (buf, sem) in enumerate(((pb0, sem_p0), (pb1, sem_p1))):
        c = it * 2 + bb
        p_desc(c, buf, sem).wait()

        def triple(s, carry2):
          b = c * CH + s
          a0 = rebuf[b, pl.ds(0, 16)]
          a1 = rebuf[b, pl.ds(16, 16)]
          for kk in range(ED // 16):
            dv = hbuf[b, pl.ds(kk * 16, 16)]
            for j in range(16):
              k = kk * 16 + j
              bc = jnp.broadcast_to(dv[j], (16,))
              pv = buf[s, pl.ds(k * RD, RD)]
              p0, p1 = plsc.unpack(pv, format=plsc.PackFormat.INTERLEAVED)
              a0 = a0 + bc * p0
              a1 = a1 + bc * p1
          obuf[b, pl.ds(0, 16)] = a0
          obuf[b, pl.ds(16, 16)] = a1
          return carry2
        lax.fori_loop(0, CH, triple, 0)

        nxt = c + 2

        @pl.when(nxt < NCH)
        def _():
          p_desc(nxt, buf, sem).start()
      return carry
    lax.fori_loop(0, NCH // 2, ring, 0)

    pltpu.sync_copy(obuf, out_hbm.at[pl.ds(base, HP)])


def _tc_norm_body(x_ref, o_ref):
  x = x_ref[...]
  o_ref[...] = jnp.sqrt(jnp.sum(x * x, axis=1))


def kernel(head, relation, tail, entity_table, relation_table, proj_table):
  head2 = head.reshape(128, 128).astype(jnp.int32)
  rel2 = relation.reshape(128, 128).astype(jnp.int32)
  tail2 = tail.reshape(128, 128).astype(jnp.int32)
  projt = (proj_table.reshape(-1, 2, 16, ED).transpose(0, 3, 2, 1)
           .reshape(-1, RD * ED).astype(jnp.bfloat16))

  sc = pl.kernel(
      _sc_body,
      out_type=jax.ShapeDtypeStruct((B, RD), jnp.float32),
      mesh=plsc.VectorSubcoreMesh(core_axis_name="c", subcore_axis_name="s"),
      compiler_params=pltpu.CompilerParams(use_tc_tiling_on_sc=False,
                                           needs_layout_passes=False),
      scratch_types=[
          pltpu.VMEM((HP, ED), jnp.float32),
          pltpu.VMEM((HP, ED), jnp.float32),
          pltpu.VMEM((HP, RD), jnp.float32),
          pltpu.VMEM((HP, RD), jnp.float32),
          pltpu.VMEM((CH, RD * ED), jnp.bfloat16),
          pltpu.VMEM((CH, RD * ED), jnp.bfloat16),
          pltpu.VMEM((2, 128), jnp.int32),
          pltpu.VMEM((2, 128), jnp.int32),
          pltpu.VMEM((2, 128), jnp.int32),
          pltpu.SemaphoreType.DMA,
          pltpu.SemaphoreType.DMA,
          pltpu.SemaphoreType.DMA,
      ],
  )
  diff = sc(head2, rel2, tail2, entity_table, relation_table, projt)

  out = pl.pallas_call(
      _tc_norm_body,
      grid=(16,),
      in_specs=[pl.BlockSpec((B // 16, RD), lambda i: (i, 0))],
      out_specs=pl.BlockSpec((B // 16,), lambda i: (i,)),
      out_shape=jax.ShapeDtypeStruct((B,), jnp.float32),
  )(diff)
  return out
